# baseline (speedup 1.0000x reference)
import jax
import jax.numpy as jnp
from jax.experimental import pallas as pl

_BLK = 512


def _fused_kernel(x_ref, ids_ref, emb_ref, w_ref, b_ref, out_ref):
    h = x_ref.shape[1]
    x = x_ref[...]
    w1 = w_ref[:, :h]
    w2 = w_ref[:, h:]
    tab = jax.lax.dot_general(
        emb_ref[...], w2, (((1,), (1,)), ((), ())),
        preferred_element_type=jnp.float32)
    m = jax.lax.dot_general(
        x, w1, (((1,), (1,)), ((), ())),
        preferred_element_type=jnp.float32)
    ids = ids_ref[...].astype(jnp.float32)
    contrib = tab[0][None, :] + ids * (tab[1] - tab[0])[None, :]
    out_ref[...] = m + contrib + b_ref[...]


def kernel(input, is_predicate_id, emb_table, W, b):
    B, S, H = input.shape
    NC, HD = W.shape
    R = B * S
    x = input.reshape(R, H)
    ids = is_predicate_id.reshape(R, 1).astype(jnp.int32)
    b2 = b.reshape(1, NC)
    grid = (R // _BLK,)
    out = pl.pallas_call(
        _fused_kernel,
        grid=grid,
        in_specs=[
            pl.BlockSpec((_BLK, H), lambda i: (i, 0)),
            pl.BlockSpec((_BLK, 1), lambda i: (i, 0)),
            pl.BlockSpec((2, HD - H), lambda i: (0, 0)),
            pl.BlockSpec((NC, HD), lambda i: (0, 0)),
            pl.BlockSpec((1, NC), lambda i: (0, 0)),
        ],
        out_specs=pl.BlockSpec((_BLK, NC), lambda i: (i, 0)),
        out_shape=jax.ShapeDtypeStruct((R, NC), jnp.float32),
    )(x, ids, emb_table, W, b2)
    return out.reshape(B, S, NC)


# speedup vs baseline: 1.2941x; 1.2941x over previous
import jax
import jax.numpy as jnp
from jax.experimental import pallas as pl

_BLK = 2048


def _fused_kernel(x_ref, ids_ref, emb_ref, w_ref, b_ref, out_ref):
    h = x_ref.shape[1]
    x = x_ref[...]
    w1 = w_ref[:, :h]
    w2 = w_ref[:, h:]
    tab = jax.lax.dot_general(
        emb_ref[...], w2, (((1,), (1,)), ((), ())),
        preferred_element_type=jnp.float32)
    m = jax.lax.dot_general(
        x, w1, (((1,), (1,)), ((), ())),
        preferred_element_type=jnp.float32)
    ids = ids_ref[...].astype(jnp.float32)
    contrib = tab[0][None, :] + ids * (tab[1] - tab[0])[None, :]
    out_ref[...] = m + contrib + b_ref[...]


def kernel(input, is_predicate_id, emb_table, W, b):
    B, S, H = input.shape
    NC, HD = W.shape
    R = B * S
    x = input.reshape(R, H)
    ids = is_predicate_id.reshape(R, 1).astype(jnp.int32)
    b2 = b.reshape(1, NC)
    grid = (R // _BLK,)
    out = pl.pallas_call(
        _fused_kernel,
        grid=grid,
        in_specs=[
            pl.BlockSpec((_BLK, H), lambda i: (i, 0)),
            pl.BlockSpec((_BLK, 1), lambda i: (i, 0)),
            pl.BlockSpec((2, HD - H), lambda i: (0, 0)),
            pl.BlockSpec((NC, HD), lambda i: (0, 0)),
            pl.BlockSpec((1, NC), lambda i: (0, 0)),
        ],
        out_specs=pl.BlockSpec((_BLK, NC), lambda i: (i, 0)),
        out_shape=jax.ShapeDtypeStruct((R, NC), jnp.float32),
    )(x, ids, emb_table, W, b2)
    return out.reshape(B, S, NC)
